# baseline (speedup 1.0000x reference)
import jax
import jax.numpy as jnp
from jax import lax
from jax.experimental import pallas as pl
from jax.experimental.pallas import tpu as pltpu
from jax.experimental.pallas import tpu_sc as plsc

N = 100000
D = 32
H = 16
E = 1600000
NCORES = 2
NTILES = 16
SUB = 128
NSUB = 4
C = SUB * NSUB
N_PAD = 100352
E_PAD = 1605632


def _lane_splat(v16, k):
  idx = jnp.full((16, 1), k, jnp.int32)
  dnums = lax.GatherDimensionNumbers(
      offset_dims=(), collapsed_slice_dims=(0,), start_index_map=(0,))
  return lax.gather(v16, idx, dnums, (1,),
                    mode=lax.GatherScatterMode.PROMISE_IN_BOUNDS)


def _make_spmm():
  e_t = E_PAD // NTILES
  nch = e_t // C
  npair = nch // 2
  nz = N_PAD // NTILES
  zc = 448
  nzrep = nz // zc

  mesh = plsc.VectorSubcoreMesh(
      core_axis_name="c", subcore_axis_name="s",
      num_cores=NCORES, num_subcores=NTILES)

  def body(ind_ref, vals_ref, tlo_ref, thi_ref, olo_ref, ohi_ref,
           acc, idx0, idx1, row0, row1, val0, val1, gath0, gath1,
           isem0, isem1, ssem, gs0, gs1, gs2, gs3):
    c = lax.axis_index("c")
    s = lax.axis_index("s")
    gsems = [gs0, gs1, gs2, gs3]
    bufs = [(idx0, row0, val0, gath0, isem0),
            (idx1, row1, val1, gath1, isem1)]

    def in_descs(gi, b):
      idx_b, row_b, val_b, _, sem = bufs[b]
      eb = s * e_t + gi * C
      ds = [pltpu.make_async_copy(vals_ref.at[pl.ds(eb, C)], val_b, sem)]
      for j in range(NSUB):
        ds.append(pltpu.make_async_copy(
            ind_ref.at[pl.ds(eb + j * SUB, SUB)], row_b.at[j], sem))
        ds.append(pltpu.make_async_copy(
            ind_ref.at[pl.ds(E + eb + j * SUB, SUB)], idx_b.at[j], sem))
      return ds

    def gath_start(b, j):
      idx_b, _, _, gath_b, _ = bufs[b]
      dst = gath_b.at[pl.ds(j * SUB, SUB)]

      @pl.when(c == 0)
      def _():
        pltpu.make_async_copy(tlo_ref.at[idx_b.at[j]], dst, gsems[j]).start()

      @pl.when(c == 1)
      def _():
        pltpu.make_async_copy(thi_ref.at[idx_b.at[j]], dst, gsems[j]).start()

    def gath_wait(b, j):
      idx_b, _, _, gath_b, _ = bufs[b]
      dst = gath_b.at[pl.ds(j * SUB, SUB)]
      pltpu.make_async_copy(tlo_ref.at[idx_b.at[j]], dst, gsems[j]).wait()

    def scat_desc(b, j):
      _, row_b, _, gath_b, _ = bufs[b]
      return pltpu.make_async_copy(gath_b.at[pl.ds(j * SUB, SUB)],
                                   acc.at[row_b.at[j]], ssem)

    def scale(b, j):
      _, _, val_b, gath_b, _ = bufs[b]

      @plsc.parallel_loop(0, SUB // 16, 1, unroll=2)
      def grp(gg):
        base = j * SUB + gg * 16
        v16 = val_b[pl.ds(base, 16)]
        for k in range(16):
          vv = _lane_splat(v16, k)
          gath_b[base + k] = gath_b[base + k] * vv

    def process(b, js):
      for j in js:
        gath_wait(b, j)
        scale(b, j)
        scat_desc(b, j).start(add=True)

    def zb(j, carry):
      gath0[j] = jnp.zeros((H,), jnp.float32)
      return carry
    lax.fori_loop(0, zc, zb, 0)
    for r in range(nzrep):
      pltpu.sync_copy(gath0.at[pl.ds(0, zc)],
                      acc.at[pl.ds(s * nz + r * zc, zc)])
    plsc.subcore_barrier()

    for d in in_descs(0, 0):
      d.start()
    for d in in_descs(0, 0):
      d.wait()
    for j in range(NSUB):
      gath_start(0, j)

    half0 = tuple(range(NSUB // 2))
    half1 = tuple(range(NSUB // 2, NSUB))

    def pair(p, carry):
      ga = 2 * p
      @pl.when(p > 0)
      def _():
        for j in range(NSUB):
          scat_desc(1, j).wait()
      for d in in_descs(ga + 1, 1):
        d.start()
      process(0, half0)
      for d in in_descs(ga + 1, 1):
        d.wait()
      for j in range(NSUB):
        gath_start(1, j)
      process(0, half1)
      for j in range(NSUB):
        scat_desc(0, j).wait()
      @pl.when(p + 1 < npair)
      def _():
        for d in in_descs(ga + 2, 0):
          d.start()
      process(1, half0)
      @pl.when(p + 1 < npair)
      def _():
        for d in in_descs(ga + 2, 0):
          d.wait()
        for j in range(NSUB):
          gath_start(0, j)
      process(1, half1)
      return carry
    lax.fori_loop(0, npair, pair, 0)

    for j in range(NSUB):
      scat_desc(1, j).wait()

    plsc.subcore_barrier()

    @pl.when(c == 0)
    def _():
      pltpu.sync_copy(acc.at[pl.ds(s * nz, nz)],
                      olo_ref.at[pl.ds(s * nz, nz)])

    @pl.when(c == 1)
    def _():
      pltpu.sync_copy(acc.at[pl.ds(s * nz, nz)],
                      ohi_ref.at[pl.ds(s * nz, nz)])

  return pl.kernel(
      body,
      out_type=(jax.ShapeDtypeStruct((N_PAD, H), jnp.float32),
                jax.ShapeDtypeStruct((N_PAD, H), jnp.float32)),
      mesh=mesh,
      compiler_params=pltpu.CompilerParams(use_tc_tiling_on_sc=False),
      scratch_types=[
          pltpu.VMEM_SHARED((N_PAD, H), jnp.float32),
          pltpu.VMEM((NSUB, SUB), jnp.int32),
          pltpu.VMEM((NSUB, SUB), jnp.int32),
          pltpu.VMEM((NSUB, SUB), jnp.int32),
          pltpu.VMEM((NSUB, SUB), jnp.int32),
          pltpu.VMEM((C,), jnp.float32),
          pltpu.VMEM((C,), jnp.float32),
          pltpu.VMEM((C, H), jnp.float32),
          pltpu.VMEM((C, H), jnp.float32),
          pltpu.SemaphoreType.DMA,
          pltpu.SemaphoreType.DMA,
          pltpu.SemaphoreType.DMA,
          pltpu.SemaphoreType.DMA,
          pltpu.SemaphoreType.DMA,
          pltpu.SemaphoreType.DMA,
          pltpu.SemaphoreType.DMA,
      ],
  )


_BM = 2000


def _mm_body(s0_ref, s1_ref, w_ref, hstd_ref, hlo_ref, hhi_ref):
  w = w_ref[...]
  s0 = s0_ref[...]
  s1 = s1_ref[...]
  x = (jnp.dot(s0, w[:H, :], preferred_element_type=jnp.float32) +
       jnp.dot(s1, w[H:, :], preferred_element_type=jnp.float32))
  h = jnp.maximum(x, 0.0)
  hstd_ref[...] = h
  hlo_ref[...] = h[:, :H]
  hhi_ref[...] = h[:, H:]


def _matmul_relu(s_lo, s_hi, w):
  grid = N // _BM
  half = pl.BlockSpec((_BM, H), lambda i: (i, 0))
  return pl.pallas_call(
      _mm_body,
      grid=(grid,),
      in_specs=[half, half, pl.BlockSpec((D, D), lambda i: (0, 0))],
      out_specs=[pl.BlockSpec((_BM, D), lambda i: (i, 0)), half, half],
      out_shape=[
          jax.ShapeDtypeStruct((N, D), jnp.float32),
          jax.ShapeDtypeStruct((N_PAD, H), jnp.float32),
          jax.ShapeDtypeStruct((N_PAD, H), jnp.float32),
      ],
  )(s_lo, s_hi, w)


def _split_body(e_ref, lo_ref, hi_ref):
  e = e_ref[...]
  lo_ref[...] = e[:, :H]
  hi_ref[...] = e[:, H:]


def _split_halves(emb):
  grid = N // _BM
  half = pl.BlockSpec((_BM, H), lambda i: (i, 0))
  return pl.pallas_call(
      _split_body,
      grid=(grid,),
      in_specs=[pl.BlockSpec((_BM, D), lambda i: (i, 0))],
      out_specs=[half, half],
      out_shape=[
          jax.ShapeDtypeStruct((N_PAD, H), jnp.float32),
          jax.ShapeDtypeStruct((N_PAD, H), jnp.float32),
      ],
  )(emb)


def _pool_body(e0_ref, e1_ref, e2_ref, w_ref, b_ref, out_ref):
  w = w_ref[...]
  b = b_ref[0, 0]
  e0 = e0_ref[...]
  e1 = e1_ref[...]
  e2 = e2_ref[...]
  a0 = jax.nn.sigmoid(jnp.sum(e0 * w, axis=1, keepdims=True) + b)
  a1 = jax.nn.sigmoid(jnp.sum(e1 * w, axis=1, keepdims=True) + b)
  a2 = jax.nn.sigmoid(jnp.sum(e2 * w, axis=1, keepdims=True) + b)
  m = jnp.maximum(jnp.maximum(a0, a1), a2)
  x0 = jnp.exp(a0 - m)
  x1 = jnp.exp(a1 - m)
  x2 = jnp.exp(a2 - m)
  inv = 1.0 / (x0 + x1 + x2)
  out_ref[...] = (e0 * x0 + e1 * x1 + e2 * x2) * inv


def _pool(e0, e1, e2, w_row, b11):
  grid = N // _BM
  blk = pl.BlockSpec((_BM, D), lambda i: (i, 0))
  return pl.pallas_call(
      _pool_body,
      grid=(grid,),
      in_specs=[blk, blk, blk,
                pl.BlockSpec((1, D), lambda i: (0, 0)),
                pl.BlockSpec((1, 1), lambda i: (0, 0))],
      out_specs=blk,
      out_shape=jax.ShapeDtypeStruct((N, D), jnp.float32),
  )(e0, e1, e2, w_row, b11)


def _branch(indices, values, emb0, w, attn_w, attn_b, spmm):
  ind_flat = indices.reshape(3 * E)
  vals = jnp.pad(values, (0, E_PAD - E))
  e0_lo, e0_hi = _split_halves(emb0)

  s0_lo, s0_hi = spmm(ind_flat, vals, e0_lo, e0_hi)
  h1_std, h1_lo, h1_hi = _matmul_relu(s0_lo, s0_hi, w)
  s1_lo, s1_hi = spmm(ind_flat, vals, h1_lo, h1_hi)
  h2_std, _, _ = _matmul_relu(s1_lo, s1_hi, w)
  return _pool(emb0, h1_std, h2_std,
               attn_w.reshape(1, D), attn_b.reshape(1, 1))


def kernel(adj_u1_indices, adj_u1_values, adj_i1_indices, adj_i1_values,
           user_emb, item_emb, W_u, W_i,
           attn_u_w, attn_u_b, attn_i_w, attn_i_b):
  spmm = _make_spmm()
  u_out = _branch(adj_u1_indices, adj_u1_values, user_emb, W_u,
                  attn_u_w, attn_u_b, spmm)
  i_out = _branch(adj_i1_indices, adj_i1_values, item_emb, W_i,
                  attn_i_w, attn_i_b, spmm)
  return (u_out, i_out)


# speedup vs baseline: 1.0017x; 1.0017x over previous
import jax
import jax.numpy as jnp
from jax import lax
from jax.experimental import pallas as pl
from jax.experimental.pallas import tpu as pltpu
from jax.experimental.pallas import tpu_sc as plsc

N = 100000
D = 32
H = 16
E = 1600000
NCORES = 2
NTILES = 16
SUB = 128
NSUB = 4
C = SUB * NSUB
N_PAD = 100352
E_PAD = 1605632


def _lane_splat(v16, k):
  idx = jnp.full((16, 1), k, jnp.int32)
  dnums = lax.GatherDimensionNumbers(
      offset_dims=(), collapsed_slice_dims=(0,), start_index_map=(0,))
  return lax.gather(v16, idx, dnums, (1,),
                    mode=lax.GatherScatterMode.PROMISE_IN_BOUNDS)


def _make_spmm():
  e_t = E_PAD // NTILES
  nch = e_t // C
  npair = nch // 2
  nz = N_PAD // NTILES
  zc = 448
  nzrep = nz // zc

  mesh = plsc.VectorSubcoreMesh(
      core_axis_name="c", subcore_axis_name="s",
      num_cores=NCORES, num_subcores=NTILES)

  def body(ind_ref, vals_ref, tlo_ref, thi_ref, olo_ref, ohi_ref,
           acc, idx0, idx1, row0, row1, val0, val1, gath0, gath1,
           isem0, isem1, ssem, gs0, gs1, gs2, gs3):
    c = lax.axis_index("c")
    s = lax.axis_index("s")
    gsems = [gs0, gs1, gs2, gs3]
    bufs = [(idx0, row0, val0, gath0, isem0),
            (idx1, row1, val1, gath1, isem1)]

    def in_descs(gi, b):
      idx_b, row_b, val_b, _, sem = bufs[b]
      eb = s * e_t + gi * C
      ds = [pltpu.make_async_copy(vals_ref.at[pl.ds(eb, C)], val_b, sem)]
      for j in range(NSUB):
        ds.append(pltpu.make_async_copy(
            ind_ref.at[pl.ds(eb + j * SUB, SUB)], row_b.at[j], sem))
        ds.append(pltpu.make_async_copy(
            ind_ref.at[pl.ds(E + eb + j * SUB, SUB)], idx_b.at[j], sem))
      return ds

    def gath_start(b, j):
      idx_b, _, _, gath_b, _ = bufs[b]
      dst = gath_b.at[pl.ds(j * SUB, SUB)]

      @pl.when(c == 0)
      def _():
        pltpu.make_async_copy(tlo_ref.at[idx_b.at[j]], dst, gsems[j]).start()

      @pl.when(c == 1)
      def _():
        pltpu.make_async_copy(thi_ref.at[idx_b.at[j]], dst, gsems[j]).start()

    def gath_wait(b, j):
      idx_b, _, _, gath_b, _ = bufs[b]
      dst = gath_b.at[pl.ds(j * SUB, SUB)]
      pltpu.make_async_copy(tlo_ref.at[idx_b.at[j]], dst, gsems[j]).wait()

    def scat_desc(b, j):
      _, row_b, _, gath_b, _ = bufs[b]
      return pltpu.make_async_copy(gath_b.at[pl.ds(j * SUB, SUB)],
                                   acc.at[row_b.at[j]], ssem)

    def scale(b, j):
      _, _, val_b, gath_b, _ = bufs[b]

      @plsc.parallel_loop(0, SUB // 16, 1, unroll=2)
      def grp(gg):
        base = j * SUB + gg * 16
        v16 = val_b[pl.ds(base, 16)]
        for k in range(16):
          vv = _lane_splat(v16, k)
          gath_b[base + k] = gath_b[base + k] * vv

    def process(b, js):
      for j in js:
        gath_wait(b, j)
        scale(b, j)
        scat_desc(b, j).start(add=True)

    def zb(j, carry):
      gath0[j] = jnp.zeros((H,), jnp.float32)
      return carry
    lax.fori_loop(0, zc, zb, 0)
    for r in range(nzrep):
      pltpu.sync_copy(gath0.at[pl.ds(0, zc)],
                      acc.at[pl.ds(s * nz + r * zc, zc)])
    plsc.subcore_barrier()

    for d in in_descs(0, 0):
      d.start()
    for d in in_descs(0, 0):
      d.wait()
    for j in range(NSUB):
      gath_start(0, j)

    half0 = tuple(range(NSUB // 2))
    half1 = tuple(range(NSUB // 2, NSUB))

    def pair(p, carry):
      ga = 2 * p
      @pl.when(p > 0)
      def _():
        for j in range(NSUB):
          scat_desc(1, j).wait()
      for d in in_descs(ga + 1, 1):
        d.start()
      process(0, half0)
      for d in in_descs(ga + 1, 1):
        d.wait()
      for j in range(NSUB):
        gath_start(1, j)
      process(0, half1)
      for j in range(NSUB):
        scat_desc(0, j).wait()
      @pl.when(p + 1 < npair)
      def _():
        for d in in_descs(ga + 2, 0):
          d.start()
      process(1, half0)
      @pl.when(p + 1 < npair)
      def _():
        for d in in_descs(ga + 2, 0):
          d.wait()
        for j in range(NSUB):
          gath_start(0, j)
      process(1, half1)
      return carry
    lax.fori_loop(0, npair, pair, 0)

    for j in range(NSUB):
      scat_desc(1, j).wait()

    plsc.subcore_barrier()

    @pl.when(c == 0)
    def _():
      pltpu.sync_copy(acc.at[pl.ds(s * nz, nz)],
                      olo_ref.at[pl.ds(s * nz, nz)])

    @pl.when(c == 1)
    def _():
      pltpu.sync_copy(acc.at[pl.ds(s * nz, nz)],
                      ohi_ref.at[pl.ds(s * nz, nz)])

  return pl.kernel(
      body,
      out_type=(jax.ShapeDtypeStruct((N_PAD, H), jnp.float32),
                jax.ShapeDtypeStruct((N_PAD, H), jnp.float32)),
      mesh=mesh,
      compiler_params=pltpu.CompilerParams(use_tc_tiling_on_sc=False),
      scratch_types=[
          pltpu.VMEM_SHARED((N_PAD, H), jnp.float32),
          pltpu.VMEM((NSUB, SUB), jnp.int32),
          pltpu.VMEM((NSUB, SUB), jnp.int32),
          pltpu.VMEM((NSUB, SUB), jnp.int32),
          pltpu.VMEM((NSUB, SUB), jnp.int32),
          pltpu.VMEM((C,), jnp.float32),
          pltpu.VMEM((C,), jnp.float32),
          pltpu.VMEM((C, H), jnp.float32),
          pltpu.VMEM((C, H), jnp.float32),
          pltpu.SemaphoreType.DMA,
          pltpu.SemaphoreType.DMA,
          pltpu.SemaphoreType.DMA,
          pltpu.SemaphoreType.DMA,
          pltpu.SemaphoreType.DMA,
          pltpu.SemaphoreType.DMA,
          pltpu.SemaphoreType.DMA,
      ],
  )


_BM = 2000


def _mm_body(s0_ref, s1_ref, w_ref, hstd_ref, hlo_ref, hhi_ref):
  w = w_ref[...]
  s0 = s0_ref[...]
  s1 = s1_ref[...]
  x = (jnp.dot(s0, w[:H, :], preferred_element_type=jnp.float32) +
       jnp.dot(s1, w[H:, :], preferred_element_type=jnp.float32))
  h = jnp.maximum(x, 0.0)
  hstd_ref[...] = h
  hlo_ref[...] = h[:, :H]
  hhi_ref[...] = h[:, H:]


def _matmul_relu(s_lo, s_hi, w):
  grid = N // _BM
  half = pl.BlockSpec((_BM, H), lambda i: (i, 0))
  return pl.pallas_call(
      _mm_body,
      grid=(grid,),
      in_specs=[half, half, pl.BlockSpec((D, D), lambda i: (0, 0))],
      out_specs=[pl.BlockSpec((_BM, D), lambda i: (i, 0)), half, half],
      out_shape=[
          jax.ShapeDtypeStruct((N, D), jnp.float32),
          jax.ShapeDtypeStruct((N_PAD, H), jnp.float32),
          jax.ShapeDtypeStruct((N_PAD, H), jnp.float32),
      ],
  )(s_lo, s_hi, w)


def _split_body(e_ref, lo_ref, hi_ref):
  e = e_ref[...]
  lo_ref[...] = e[:, :H]
  hi_ref[...] = e[:, H:]


def _split_halves(emb):
  grid = N // _BM
  half = pl.BlockSpec((_BM, H), lambda i: (i, 0))
  return pl.pallas_call(
      _split_body,
      grid=(grid,),
      in_specs=[pl.BlockSpec((_BM, D), lambda i: (i, 0))],
      out_specs=[half, half],
      out_shape=[
          jax.ShapeDtypeStruct((N_PAD, H), jnp.float32),
          jax.ShapeDtypeStruct((N_PAD, H), jnp.float32),
      ],
  )(emb)


def _pool_body(e0_ref, e1_ref, e2_ref, w_ref, b_ref, out_ref):
  w = w_ref[...]
  b = b_ref[0, 0]
  e0 = e0_ref[...]
  e1 = e1_ref[...]
  e2 = e2_ref[...]
  a0 = jax.nn.sigmoid(jnp.sum(e0 * w, axis=1, keepdims=True) + b)
  a1 = jax.nn.sigmoid(jnp.sum(e1 * w, axis=1, keepdims=True) + b)
  a2 = jax.nn.sigmoid(jnp.sum(e2 * w, axis=1, keepdims=True) + b)
  m = jnp.maximum(jnp.maximum(a0, a1), a2)
  x0 = jnp.exp(a0 - m)
  x1 = jnp.exp(a1 - m)
  x2 = jnp.exp(a2 - m)
  inv = 1.0 / (x0 + x1 + x2)
  out_ref[...] = (e0 * x0 + e1 * x1 + e2 * x2) * inv


def _pool(e0, e1, e2, w_row, b11):
  grid = N // _BM
  blk = pl.BlockSpec((_BM, D), lambda i: (i, 0))
  return pl.pallas_call(
      _pool_body,
      grid=(grid,),
      in_specs=[blk, blk, blk,
                pl.BlockSpec((1, D), lambda i: (0, 0)),
                pl.BlockSpec((1, 1), lambda i: (0, 0))],
      out_specs=blk,
      out_shape=jax.ShapeDtypeStruct((N, D), jnp.float32),
  )(e0, e1, e2, w_row, b11)


def kernel(adj_u1_indices, adj_u1_values, adj_i1_indices, adj_i1_values,
           user_emb, item_emb, W_u, W_i,
           attn_u_w, attn_u_b, attn_i_w, attn_i_b):
  spmm = _make_spmm()

  ind_u = adj_u1_indices.reshape(3 * E)
  ind_i = adj_i1_indices.reshape(3 * E)
  vals_u = jnp.pad(adj_u1_values, (0, E_PAD - E))
  vals_i = jnp.pad(adj_i1_values, (0, E_PAD - E))
  u0_lo, u0_hi = _split_halves(user_emb)
  i0_lo, i0_hi = _split_halves(item_emb)

  su0_lo, su0_hi = spmm(ind_u, vals_u, u0_lo, u0_hi)
  si0_lo, si0_hi = spmm(ind_i, vals_i, i0_lo, i0_hi)
  hu1_std, hu1_lo, hu1_hi = _matmul_relu(su0_lo, su0_hi, W_u)
  hi1_std, hi1_lo, hi1_hi = _matmul_relu(si0_lo, si0_hi, W_i)
  su1_lo, su1_hi = spmm(ind_u, vals_u, hu1_lo, hu1_hi)
  si1_lo, si1_hi = spmm(ind_i, vals_i, hi1_lo, hi1_hi)
  hu2_std, _, _ = _matmul_relu(su1_lo, su1_hi, W_u)
  hi2_std, _, _ = _matmul_relu(si1_lo, si1_hi, W_i)
  u_out = _pool(user_emb, hu1_std, hu2_std,
                attn_u_w.reshape(1, D), attn_u_b.reshape(1, 1))
  i_out = _pool(item_emb, hi1_std, hi2_std,
                attn_i_w.reshape(1, D), attn_i_b.reshape(1, 1))
  return (u_out, i_out)


# speedup vs baseline: 1.2480x; 1.2459x over previous
import jax
import jax.numpy as jnp
from jax import lax
from jax.experimental import pallas as pl
from jax.experimental.pallas import tpu as pltpu
from jax.experimental.pallas import tpu_sc as plsc

N = 100000
D = 32
H = 16
E = 1600000
NCORES = 2
NTILES = 16
SUB = 128
NSUB = 4
C = SUB * NSUB
N_PAD = 100352
E_PAD = 1605632


def _lane_splat(v16, k):
  idx = jnp.full((16, 1), k, jnp.int32)
  dnums = lax.GatherDimensionNumbers(
      offset_dims=(), collapsed_slice_dims=(0,), start_index_map=(0,))
  return lax.gather(v16, idx, dnums, (1,),
                    mode=lax.GatherScatterMode.PROMISE_IN_BOUNDS)


def _make_spmm():
  e_t = E_PAD // NTILES
  nch = e_t // C
  npair = nch // 2
  nz = N_PAD // NTILES
  zc = 448
  nzrep = nz // zc

  mesh = plsc.VectorSubcoreMesh(
      core_axis_name="c", subcore_axis_name="s",
      num_cores=NCORES, num_subcores=NTILES)

  def body(rows_ref, cols_ref, vals_ref, tlo_ref, thi_ref, olo_ref, ohi_ref,
           acc, idx0, idx1, row0, row1, val0, val1, gath0, gath1,
           isem0, isem1, ssem, gs0, gs1, gs2, gs3):
    c = lax.axis_index("c")
    s = lax.axis_index("s")
    gsems = [gs0, gs1, gs2, gs3]
    bufs = [(idx0, row0, val0, gath0, isem0),
            (idx1, row1, val1, gath1, isem1)]

    def in_descs(gi, b):
      idx_b, row_b, val_b, _, sem = bufs[b]
      eb = s * e_t + gi * C
      ds = [pltpu.make_async_copy(vals_ref.at[pl.ds(eb, C)], val_b, sem)]
      for j in range(NSUB):
        ds.append(pltpu.make_async_copy(
            rows_ref.at[pl.ds(eb + j * SUB, SUB)], row_b.at[j], sem))
        ds.append(pltpu.make_async_copy(
            cols_ref.at[pl.ds(eb + j * SUB, SUB)], idx_b.at[j], sem))
      return ds

    def gath_start(b, j):
      idx_b, _, _, gath_b, _ = bufs[b]
      dst = gath_b.at[pl.ds(j * SUB, SUB)]

      @pl.when(c == 0)
      def _():
        pltpu.make_async_copy(tlo_ref.at[idx_b.at[j]], dst, gsems[j]).start()

      @pl.when(c == 1)
      def _():
        pltpu.make_async_copy(thi_ref.at[idx_b.at[j]], dst, gsems[j]).start()

    def gath_wait(b, j):
      idx_b, _, _, gath_b, _ = bufs[b]
      dst = gath_b.at[pl.ds(j * SUB, SUB)]
      pltpu.make_async_copy(tlo_ref.at[idx_b.at[j]], dst, gsems[j]).wait()

    def scat_desc(b, j):
      _, row_b, _, gath_b, _ = bufs[b]
      return pltpu.make_async_copy(gath_b.at[pl.ds(j * SUB, SUB)],
                                   acc.at[row_b.at[j]], ssem)

    def scale(b, j):
      _, _, val_b, gath_b, _ = bufs[b]

      @plsc.parallel_loop(0, SUB // 16, 1, unroll=2)
      def grp(gg):
        base = j * SUB + gg * 16
        v16 = val_b[pl.ds(base, 16)]
        for k in range(16):
          vv = _lane_splat(v16, k)
          gath_b[base + k] = gath_b[base + k] * vv

    def process(b, js):
      for j in js:
        gath_wait(b, j)
        scale(b, j)
        scat_desc(b, j).start(add=True)

    def zb(j, carry):
      gath0[j] = jnp.zeros((H,), jnp.float32)
      return carry
    lax.fori_loop(0, zc, zb, 0)
    for r in range(nzrep):
      pltpu.sync_copy(gath0.at[pl.ds(0, zc)],
                      acc.at[pl.ds(s * nz + r * zc, zc)])
    plsc.subcore_barrier()

    for d in in_descs(0, 0):
      d.start()
    for d in in_descs(0, 0):
      d.wait()
    for j in range(NSUB):
      gath_start(0, j)

    half0 = tuple(range(NSUB // 2))
    half1 = tuple(range(NSUB // 2, NSUB))

    def pair(p, carry):
      ga = 2 * p
      @pl.when(p > 0)
      def _():
        for j in range(NSUB):
          scat_desc(1, j).wait()
      for d in in_descs(ga + 1, 1):
        d.start()
      process(0, half0)
      for d in in_descs(ga + 1, 1):
        d.wait()
      for j in range(NSUB):
        gath_start(1, j)
      process(0, half1)
      for j in range(NSUB):
        scat_desc(0, j).wait()
      @pl.when(p + 1 < npair)
      def _():
        for d in in_descs(ga + 2, 0):
          d.start()
      process(1, half0)
      @pl.when(p + 1 < npair)
      def _():
        for d in in_descs(ga + 2, 0):
          d.wait()
        for j in range(NSUB):
          gath_start(0, j)
      process(1, half1)
      return carry
    lax.fori_loop(0, npair, pair, 0)

    for j in range(NSUB):
      scat_desc(1, j).wait()

    plsc.subcore_barrier()

    @pl.when(c == 0)
    def _():
      pltpu.sync_copy(acc.at[pl.ds(s * nz, nz)],
                      olo_ref.at[pl.ds(s * nz, nz)])

    @pl.when(c == 1)
    def _():
      pltpu.sync_copy(acc.at[pl.ds(s * nz, nz)],
                      ohi_ref.at[pl.ds(s * nz, nz)])

  return pl.kernel(
      body,
      out_type=(jax.ShapeDtypeStruct((N_PAD, H), jnp.float32),
                jax.ShapeDtypeStruct((N_PAD, H), jnp.float32)),
      mesh=mesh,
      compiler_params=pltpu.CompilerParams(use_tc_tiling_on_sc=False),
      scratch_types=[
          pltpu.VMEM_SHARED((N_PAD, H), jnp.float32),
          pltpu.VMEM((NSUB, SUB), jnp.int32),
          pltpu.VMEM((NSUB, SUB), jnp.int32),
          pltpu.VMEM((NSUB, SUB), jnp.int32),
          pltpu.VMEM((NSUB, SUB), jnp.int32),
          pltpu.VMEM((C,), jnp.float32),
          pltpu.VMEM((C,), jnp.float32),
          pltpu.VMEM((C, H), jnp.float32),
          pltpu.VMEM((C, H), jnp.float32),
          pltpu.SemaphoreType.DMA,
          pltpu.SemaphoreType.DMA,
          pltpu.SemaphoreType.DMA,
          pltpu.SemaphoreType.DMA,
          pltpu.SemaphoreType.DMA,
          pltpu.SemaphoreType.DMA,
          pltpu.SemaphoreType.DMA,
      ],
  )


_BM = 2000


def _mm_body(s0_ref, s1_ref, w_ref, hstd_ref, hlo_ref, hhi_ref):
  w = w_ref[...]
  s0 = s0_ref[...]
  s1 = s1_ref[...]
  x = (jnp.dot(s0, w[:H, :], preferred_element_type=jnp.float32) +
       jnp.dot(s1, w[H:, :], preferred_element_type=jnp.float32))
  h = jnp.maximum(x, 0.0)
  hstd_ref[...] = h
  hlo_ref[...] = h[:, :H]
  hhi_ref[...] = h[:, H:]


def _matmul_relu(s_lo, s_hi, w):
  grid = N // _BM
  half = pl.BlockSpec((_BM, H), lambda i: (i, 0))
  return pl.pallas_call(
      _mm_body,
      grid=(grid,),
      in_specs=[half, half, pl.BlockSpec((D, D), lambda i: (0, 0))],
      out_specs=[pl.BlockSpec((_BM, D), lambda i: (i, 0)), half, half],
      out_shape=[
          jax.ShapeDtypeStruct((N, D), jnp.float32),
          jax.ShapeDtypeStruct((N_PAD, H), jnp.float32),
          jax.ShapeDtypeStruct((N_PAD, H), jnp.float32),
      ],
  )(s_lo, s_hi, w)


def _split_body(e_ref, lo_ref, hi_ref):
  e = e_ref[...]
  lo_ref[...] = e[:, :H]
  hi_ref[...] = e[:, H:]


def _split_halves(emb):
  grid = N // _BM
  half = pl.BlockSpec((_BM, H), lambda i: (i, 0))
  return pl.pallas_call(
      _split_body,
      grid=(grid,),
      in_specs=[pl.BlockSpec((_BM, D), lambda i: (i, 0))],
      out_specs=[half, half],
      out_shape=[
          jax.ShapeDtypeStruct((N_PAD, H), jnp.float32),
          jax.ShapeDtypeStruct((N_PAD, H), jnp.float32),
      ],
  )(emb)


def _pool_body(e0_ref, e1_ref, e2_ref, w_ref, b_ref, out_ref):
  w = w_ref[...]
  b = b_ref[0, 0]
  e0 = e0_ref[...]
  e1 = e1_ref[...]
  e2 = e2_ref[...]
  a0 = jax.nn.sigmoid(jnp.sum(e0 * w, axis=1, keepdims=True) + b)
  a1 = jax.nn.sigmoid(jnp.sum(e1 * w, axis=1, keepdims=True) + b)
  a2 = jax.nn.sigmoid(jnp.sum(e2 * w, axis=1, keepdims=True) + b)
  m = jnp.maximum(jnp.maximum(a0, a1), a2)
  x0 = jnp.exp(a0 - m)
  x1 = jnp.exp(a1 - m)
  x2 = jnp.exp(a2 - m)
  inv = 1.0 / (x0 + x1 + x2)
  out_ref[...] = (e0 * x0 + e1 * x1 + e2 * x2) * inv


def _pool(e0, e1, e2, w_row, b11):
  grid = N // _BM
  blk = pl.BlockSpec((_BM, D), lambda i: (i, 0))
  return pl.pallas_call(
      _pool_body,
      grid=(grid,),
      in_specs=[blk, blk, blk,
                pl.BlockSpec((1, D), lambda i: (0, 0)),
                pl.BlockSpec((1, 1), lambda i: (0, 0))],
      out_specs=blk,
      out_shape=jax.ShapeDtypeStruct((N, D), jnp.float32),
  )(e0, e1, e2, w_row, b11)


def kernel(adj_u1_indices, adj_u1_values, adj_i1_indices, adj_i1_values,
           user_emb, item_emb, W_u, W_i,
           attn_u_w, attn_u_b, attn_i_w, attn_i_b):
  spmm = _make_spmm()

  rows_u = jnp.pad(adj_u1_indices[0], (0, E_PAD - E))
  cols_u = jnp.pad(adj_u1_indices[1], (0, E_PAD - E))
  rows_i = jnp.pad(adj_i1_indices[0], (0, E_PAD - E))
  cols_i = jnp.pad(adj_i1_indices[1], (0, E_PAD - E))
  vals_u = jnp.pad(adj_u1_values, (0, E_PAD - E))
  vals_i = jnp.pad(adj_i1_values, (0, E_PAD - E))
  u0_lo, u0_hi = _split_halves(user_emb)
  i0_lo, i0_hi = _split_halves(item_emb)

  su0_lo, su0_hi = spmm(rows_u, cols_u, vals_u, u0_lo, u0_hi)
  si0_lo, si0_hi = spmm(rows_i, cols_i, vals_i, i0_lo, i0_hi)
  hu1_std, hu1_lo, hu1_hi = _matmul_relu(su0_lo, su0_hi, W_u)
  hi1_std, hi1_lo, hi1_hi = _matmul_relu(si0_lo, si0_hi, W_i)
  su1_lo, su1_hi = spmm(rows_u, cols_u, vals_u, hu1_lo, hu1_hi)
  si1_lo, si1_hi = spmm(rows_i, cols_i, vals_i, hi1_lo, hi1_hi)
  hu2_std, _, _ = _matmul_relu(su1_lo, su1_hi, W_u)
  hi2_std, _, _ = _matmul_relu(si1_lo, si1_hi, W_i)
  u_out = _pool(user_emb, hu1_std, hu2_std,
                attn_u_w.reshape(1, D), attn_u_b.reshape(1, 1))
  i_out = _pool(item_emb, hi1_std, hi2_std,
                attn_i_w.reshape(1, D), attn_i_b.reshape(1, 1))
  return (u_out, i_out)
